# initial kernel scaffold (unmeasured)
import jax
import jax.numpy as jnp
from jax import lax
from jax.experimental import pallas as pl
from jax.experimental.pallas import tpu as pltpu

B, NB, BS, H, D = 8, 64, 16, 8, 64
P_LOCAL = 64
T = P_LOCAL * BS
NEG = -1e30


def kernel(Q, K, V, bt, lens):
    lens2d = lens.reshape(B, 1)

    def body(q_ref, k_ref, v_ref, bt_ref, lens_ref, out_ref,
             send_o, recv_o, send_s, recv_s, send_sems, recv_sems):
        my_x = lax.axis_index("x")
        my_y = lax.axis_index("y")
        nbr = (my_x, 1 - my_y)

        barrier = pltpu.get_barrier_semaphore()
        pl.semaphore_signal(barrier, inc=1, device_id=nbr,
                            device_id_type=pl.DeviceIdType.MESH)
        pl.semaphore_wait(barrier, 1)

        bt_v = bt_ref[:, :]
        slot = lax.broadcasted_iota(jnp.int32, (B, NB), 1)
        valid = slot < lens_ref[:, :]
        page = (lax.broadcasted_iota(jnp.int32, (B, NB, P_LOCAL), 2)
                + my_y * P_LOCAL)
        match = (bt_v[:, :, None] == page) & valid[:, :, None]
        counts = jnp.sum(match.astype(jnp.float32), axis=1)
        counts_t = jnp.broadcast_to(
            counts[:, :, None], (B, P_LOCAL, BS)).reshape(B, T)
        has = counts_t > 0.0

        q = q_ref[:, 0, :, :].astype(jnp.bfloat16)
        k = k_ref[...].reshape(T, H, D).astype(jnp.bfloat16)
        v = v_ref[...].reshape(T, H, D).astype(jnp.bfloat16)
        scale = D ** -0.5

        m_cols, l_cols = [], []
        for h in range(H):
            s_h = lax.dot_general(
                q[:, h, :], k[:, h, :], (((1,), (1,)), ((), ())),
                preferred_element_type=jnp.float32) * scale
            s_h = jnp.where(has, s_h, NEG)
            m_h = jnp.max(s_h, axis=1, keepdims=True)
            p_h = counts_t * jnp.exp(s_h - m_h)
            l_h = jnp.sum(p_h, axis=1, keepdims=True)
            o_h = lax.dot_general(
                p_h.astype(jnp.bfloat16), v[:, h, :],
                (((1,), (0,)), ((), ())),
                preferred_element_type=jnp.float32)
            send_o[:, h, :] = o_h
            m_cols.append(m_h)
            l_cols.append(l_h)

        m_l = jnp.concatenate(m_cols, axis=1)
        l_l = jnp.concatenate(l_cols, axis=1)
        send_s[0, :, :] = m_l
        send_s[1, :, :] = l_l
        o_l = send_o[...]

        copy_o = pltpu.make_async_remote_copy(
            src_ref=send_o, dst_ref=recv_o,
            send_sem=send_sems.at[0], recv_sem=recv_sems.at[0],
            device_id=nbr, device_id_type=pl.DeviceIdType.MESH)
        copy_s = pltpu.make_async_remote_copy(
            src_ref=send_s, dst_ref=recv_s,
            send_sem=send_sems.at[1], recv_sem=recv_sems.at[1],
            device_id=nbr, device_id_type=pl.DeviceIdType.MESH)
        copy_o.start()
        copy_s.start()
        copy_o.wait()
        copy_s.wait()

        m_o = recv_s[0, :, :]
        l_o = recv_s[1, :, :]
        o_o = recv_o[...]
        m_g = jnp.maximum(m_l, m_o)
        a = jnp.exp(m_l - m_g)
        b = jnp.exp(m_o - m_g)
        l_g = l_l * a + l_o * b
        out = (o_l * a[:, :, None] + o_o * b[:, :, None]) / l_g[:, :, None]
        out_ref[:, 0, :, :] = out

    return pl.pallas_call(
        body,
        out_shape=jax.ShapeDtypeStruct((B, 1, H, D), jnp.float32),
        in_specs=[pl.BlockSpec(memory_space=pltpu.VMEM)] * 5,
        out_specs=pl.BlockSpec(memory_space=pltpu.VMEM),
        scratch_shapes=[
            pltpu.VMEM((B, H, D), jnp.float32),
            pltpu.VMEM((B, H, D), jnp.float32),
            pltpu.VMEM((2, B, H), jnp.float32),
            pltpu.VMEM((2, B, H), jnp.float32),
            pltpu.SemaphoreType.DMA((2,)),
            pltpu.SemaphoreType.DMA((2,)),
        ],
        compiler_params=pltpu.CompilerParams(collective_id=0),
    )(Q, K, V, bt, lens2d)


# baseline (device time: 61321 ns/iter reference)
import jax
import jax.numpy as jnp
from jax import lax
from jax.experimental import pallas as pl
from jax.experimental.pallas import tpu as pltpu

B, NB, BS, H, D = 8, 64, 16, 8, 64
P_LOCAL = 64
T = P_LOCAL * BS
NEG = -1e30


def kernel(Q, K, V, bt, lens):
    lens2d = lens.reshape(B, 1)

    def body(q_ref, k_ref, v_ref, bt_ref, lens_ref, out_ref,
             send_o, recv_o, send_s, recv_s, send_sems, recv_sems):
        my_x = lax.axis_index("x")
        my_y = lax.axis_index("y")
        nbr = (my_x, 1 - my_y)

        barrier = pltpu.get_barrier_semaphore()
        pl.semaphore_signal(barrier, inc=1, device_id=nbr,
                            device_id_type=pl.DeviceIdType.MESH)
        pl.semaphore_wait(barrier, 1)

        bt_v = bt_ref[:, :]
        slot = lax.broadcasted_iota(jnp.int32, (B, NB), 1)
        bt_eff = jnp.where(slot < lens_ref[:, :], bt_v, -1)
        bt_b = lax.broadcast_in_dim(bt_eff, (B, T, NB), (0, 2))
        page_of_t = (lax.broadcasted_iota(jnp.int32, (B, T, NB), 1) // BS
                     + my_y * P_LOCAL)
        counts_t = jnp.sum((bt_b == page_of_t).astype(jnp.float32),
                           axis=2)
        has = counts_t > 0.0

        scale = D ** -0.5
        o_loc, m_loc, l_loc = [], [], []
        for h in range(H):
            q_h = q_ref[:, 0, h, :].astype(jnp.bfloat16)
            k_h = k_ref[:, :, h, :].reshape(T, D).astype(jnp.bfloat16)
            v_h = v_ref[:, :, h, :].reshape(T, D).astype(jnp.bfloat16)
            s_h = lax.dot_general(
                q_h, k_h, (((1,), (1,)), ((), ())),
                preferred_element_type=jnp.float32) * scale
            s_h = jnp.where(has, s_h, NEG)
            m_h = jnp.max(s_h, axis=1, keepdims=True)
            p_h = counts_t * jnp.exp(s_h - m_h)
            l_h = jnp.sum(p_h, axis=1, keepdims=True)
            o_h = lax.dot_general(
                p_h.astype(jnp.bfloat16), v_h,
                (((1,), (0,)), ((), ())),
                preferred_element_type=jnp.float32)
            send_o[:, h, :] = o_h
            send_s[0, :, h:h + 1] = m_h
            send_s[1, :, h:h + 1] = l_h
            o_loc.append(o_h)
            m_loc.append(m_h)
            l_loc.append(l_h)

        copy_o = pltpu.make_async_remote_copy(
            src_ref=send_o, dst_ref=recv_o,
            send_sem=send_sems.at[0], recv_sem=recv_sems.at[0],
            device_id=nbr, device_id_type=pl.DeviceIdType.MESH)
        copy_s = pltpu.make_async_remote_copy(
            src_ref=send_s, dst_ref=recv_s,
            send_sem=send_sems.at[1], recv_sem=recv_sems.at[1],
            device_id=nbr, device_id_type=pl.DeviceIdType.MESH)
        copy_o.start()
        copy_s.start()
        copy_o.wait()
        copy_s.wait()

        for h in range(H):
            m_l, l_l, o_l = m_loc[h], l_loc[h], o_loc[h]
            m_o = recv_s[0, :, h:h + 1]
            l_o = recv_s[1, :, h:h + 1]
            o_o = recv_o[:, h, :]
            m_g = jnp.maximum(m_l, m_o)
            a = jnp.exp(m_l - m_g)
            b = jnp.exp(m_o - m_g)
            l_g = l_l * a + l_o * b
            out_ref[:, 0, h, :] = (o_l * a + o_o * b) / l_g

    return pl.pallas_call(
        body,
        out_shape=jax.ShapeDtypeStruct((B, 1, H, D), jnp.float32),
        in_specs=[pl.BlockSpec(memory_space=pltpu.VMEM)] * 5,
        out_specs=pl.BlockSpec(memory_space=pltpu.VMEM),
        scratch_shapes=[
            pltpu.VMEM((B, H, D), jnp.float32),
            pltpu.VMEM((B, H, D), jnp.float32),
            pltpu.VMEM((2, B, H), jnp.float32),
            pltpu.VMEM((2, B, H), jnp.float32),
            pltpu.SemaphoreType.DMA((2,)),
            pltpu.SemaphoreType.DMA((2,)),
        ],
        compiler_params=pltpu.CompilerParams(collective_id=0),
    )(Q, K, V, bt, lens2d)
